# Initial kernel scaffold; baseline (speedup 1.0000x reference)
import jax
import jax.numpy as jnp
from jax.experimental import pallas as pl


def kernel(x, table):
    raise NotImplementedError("write your pallas kernel here")


# speedup vs baseline: 2.7277x; 2.7277x over previous
import jax
import jax.numpy as jnp
from jax.experimental import pallas as pl

_BLOCK_ROWS = 512


def _copy_block(table_ref, out_ref):
    out_ref[...] = table_ref[...]


def kernel(x, table):
    seq_len = x.shape[1]
    d_model = table.shape[1]
    out = pl.pallas_call(
        _copy_block,
        grid=(seq_len // _BLOCK_ROWS,),
        in_specs=[pl.BlockSpec((_BLOCK_ROWS, d_model), lambda i: (i, 0))],
        out_specs=pl.BlockSpec((_BLOCK_ROWS, d_model), lambda i: (i, 0)),
        out_shape=jax.ShapeDtypeStruct((seq_len, d_model), table.dtype),
    )(table)
    return out[None, :, :]
